# baseline (speedup 1.0000x reference)
import functools
import math

import jax
import jax.numpy as jnp
from jax import lax
from jax.experimental import pallas as pl
from jax.experimental.pallas import tpu as pltpu
from jax.experimental.pallas import tpu_sc as plsc

D_MODEL = 768
LANES = 16
SCALE = math.sqrt(float(D_MODEL))

_B = 4 * 2048
_NW = 32
_BPW = _B // _NW
_CHUNK = 32
_NCHUNK = _BPW // _CHUNK
_NBUF = 5
_AHEAD = 3
_SUB = 8
_NSUB = _CHUNK // _SUB


def _emb_body(x_hbm, table_hbm, out_hbm, idx_v, rows_v,
              in_sem0, in_sem1, in_sem2, in_sem3, in_sem4,
              out_sem0, out_sem1, out_sem2, out_sem3, out_sem4):
    in_sems = (in_sem0, in_sem1, in_sem2, in_sem3, in_sem4)
    out_sems = (out_sem0, out_sem1, out_sem2, out_sem3, out_sem4)
    wid = lax.axis_index("s") * 2 + lax.axis_index("c")
    base = wid * _BPW
    scale = jnp.full((LANES,), SCALE, dtype=jnp.float32)

    wpr = 2048 // _BPW
    row = wid // wpr
    col = (wid % wpr) * _BPW
    pltpu.sync_copy(x_hbm.at[row, pl.ds(col, _BPW)], idx_v)

    gathers = [None] * _NCHUNK
    stores = [[None] * _NSUB for _ in range(_NCHUNK)]

    def idx_chunk(g):
        return idx_v.at[pl.ds(g * _CHUNK, _CHUNK)]

    for g in range(_AHEAD):
        gathers[g] = pltpu.async_copy(
            table_hbm.at[idx_chunk(g)], rows_v.at[g], in_sems[g])

    for g in range(_NCHUNK):
        b = g % _NBUF
        gathers[g].wait()
        nxt = g + _AHEAD
        if nxt < _NCHUNK:
            nb = nxt % _NBUF
            owner = nxt - _NBUF
            if owner >= 0:
                for st in stores[owner]:
                    st.wait()
            gathers[nxt] = pltpu.async_copy(
                table_hbm.at[idx_chunk(nxt)], rows_v.at[nb], in_sems[nb])

        buf = rows_v.at[b]
        for s in range(_NSUB):
            def body(r, carry):
                for j in range(D_MODEL // LANES):
                    sl = pl.ds(j * LANES, LANES)
                    buf[r, sl] = buf[r, sl] * scale
                return carry

            lax.fori_loop(s * _SUB, (s + 1) * _SUB, body, 0)
            stores[g][s] = pltpu.async_copy(
                buf.at[pl.ds(s * _SUB, _SUB)],
                out_hbm.at[pl.ds(base + g * _CHUNK + s * _SUB, _SUB)],
                out_sems[b])

    for g in range(_NCHUNK - _NBUF, _NCHUNK):
        if g >= 0:
            for st in stores[g]:
                st.wait()


def kernel(x, table):
    mesh = plsc.VectorSubcoreMesh(core_axis_name="c", subcore_axis_name="s")
    run = functools.partial(
        pl.kernel,
        mesh=mesh,
        compiler_params=pltpu.CompilerParams(
            disable_bounds_checks=True,
            disable_semaphore_checks=True,
            skip_device_barrier=True,
        ),
        out_type=jax.ShapeDtypeStruct((_B, D_MODEL), jnp.float32),
        scratch_types=[
            pltpu.VMEM((_BPW,), jnp.int32),
            pltpu.VMEM((_NBUF, _CHUNK, D_MODEL), jnp.float32),
        ] + [pltpu.SemaphoreType.DMA] * (2 * _NBUF),
    )(_emb_body)
    out = run(x, table)
    return out.reshape(x.shape[0], x.shape[1], D_MODEL)


# speedup vs baseline: 1.1360x; 1.1360x over previous
import functools
import math

import jax
import jax.numpy as jnp
from jax import lax
from jax.experimental import pallas as pl
from jax.experimental.pallas import tpu as pltpu
from jax.experimental.pallas import tpu_sc as plsc

D_MODEL = 768
LANES = 16
SCALE = math.sqrt(float(D_MODEL))

_B = 4 * 2048
_NW = 32
_BPW = _B // _NW
_CHUNK = 32
_NCHUNK = _BPW // _CHUNK
_NBUF = 5
_AHEAD = 3
_SUB = 32
_NSUB = _CHUNK // _SUB


def _emb_body(x_hbm, table_hbm, out_hbm, idx_v, rows_v,
              in_sem0, in_sem1, in_sem2, in_sem3, in_sem4,
              out_sem0, out_sem1, out_sem2, out_sem3, out_sem4):
    in_sems = (in_sem0, in_sem1, in_sem2, in_sem3, in_sem4)
    out_sems = (out_sem0, out_sem1, out_sem2, out_sem3, out_sem4)
    wid = lax.axis_index("s") * 2 + lax.axis_index("c")
    base = wid * _BPW
    scale = jnp.full((LANES,), SCALE, dtype=jnp.float32)

    wpr = 2048 // _BPW
    row = wid // wpr
    col = (wid % wpr) * _BPW
    pltpu.sync_copy(x_hbm.at[row, pl.ds(col, _BPW)], idx_v)

    gathers = [None] * _NCHUNK
    stores = [[None] * _NSUB for _ in range(_NCHUNK)]

    def idx_chunk(g):
        return idx_v.at[pl.ds(g * _CHUNK, _CHUNK)]

    for g in range(_AHEAD):
        gathers[g] = pltpu.async_copy(
            table_hbm.at[idx_chunk(g)], rows_v.at[g], in_sems[g])

    for g in range(_NCHUNK):
        b = g % _NBUF
        gathers[g].wait()
        nxt = g + _AHEAD
        if nxt < _NCHUNK:
            nb = nxt % _NBUF
            owner = nxt - _NBUF
            if owner >= 0:
                for st in stores[owner]:
                    st.wait()
            gathers[nxt] = pltpu.async_copy(
                table_hbm.at[idx_chunk(nxt)], rows_v.at[nb], in_sems[nb])

        buf = rows_v.at[b]
        for s in range(_NSUB):
            def body(r, carry):
                for j in range(D_MODEL // LANES):
                    sl = pl.ds(j * LANES, LANES)
                    buf[r, sl] = buf[r, sl] * scale
                return carry

            lax.fori_loop(s * _SUB, (s + 1) * _SUB, body, 0)
            stores[g][s] = pltpu.async_copy(
                buf.at[pl.ds(s * _SUB, _SUB)],
                out_hbm.at[pl.ds(base + g * _CHUNK + s * _SUB, _SUB)],
                out_sems[b])

    for g in range(_NCHUNK - _NBUF, _NCHUNK):
        if g >= 0:
            for st in stores[g]:
                st.wait()


def kernel(x, table):
    mesh = plsc.VectorSubcoreMesh(core_axis_name="c", subcore_axis_name="s")
    run = functools.partial(
        pl.kernel,
        mesh=mesh,
        compiler_params=pltpu.CompilerParams(
            disable_bounds_checks=True,
            disable_semaphore_checks=True,
            skip_device_barrier=True,
        ),
        out_type=jax.ShapeDtypeStruct((_B, D_MODEL), jnp.float32),
        scratch_types=[
            pltpu.VMEM((_BPW,), jnp.int32),
            pltpu.VMEM((_NBUF, _CHUNK, D_MODEL), jnp.float32),
        ] + [pltpu.SemaphoreType.DMA] * (2 * _NBUF),
    )(_emb_body)
    out = run(x, table)
    return out.reshape(x.shape[0], x.shape[1], D_MODEL)
